# baseline (speedup 1.0000x reference)
import functools

import jax
import jax.numpy as jnp
from jax import lax
from jax.experimental import pallas as pl
from jax.experimental.pallas import tpu as pltpu
from jax.experimental.pallas import tpu_sc as plsc

_N = 1000012
_SP = 1000064
_B = 16384
_F = 26
_K = 16
_NIDX = _B * _F
_NC, _NS = 2, 16
_NW = _NC * _NS
_RW = _B // _NW
_RCH = 128
_CH = _RCH * _F
_NSTEP = _RW // _RCH
_PER_W = _RW * _F

_CC = 55552
_NBIG = _N // _CC
_REM = _N - _NBIG * _CC
_TPP = 20
_TPW = _K * _TPP // _NW


def _sc_relayout(emb_t, tailp):
    mesh = plsc.VectorSubcoreMesh(core_axis_name="c", subcore_axis_name="s")

    @functools.partial(
        pl.kernel,
        mesh=mesh,
        out_type=jax.ShapeDtypeStruct((_K * _SP,), jnp.float32),
        scratch_types=[
            pltpu.VMEM((_CC,), jnp.float32),
            pltpu.VMEM((128,), jnp.float32),
        ],
    )
    def k(et_hbm, tl_hbm, lin_hbm, buf, rbuf):
        wid = lax.axis_index("s") * _NC + lax.axis_index("c")
        for i in range(_TPW):
            t = wid * _TPW + i
            kk = t // _TPP
            sub = t % _TPP

            @pl.when(sub < _NBIG)
            def _():
                off = sub * _CC
                pltpu.sync_copy(et_hbm.at[kk].at[pl.ds(off, _CC)], buf)
                pltpu.sync_copy(buf, lin_hbm.at[pl.ds(kk * _SP + off, _CC)])

            @pl.when(sub == _NBIG)
            def _():
                off = _NBIG * _CC
                pltpu.sync_copy(tl_hbm.at[pl.ds(kk * 128, 128)], rbuf)
                pltpu.sync_copy(rbuf, lin_hbm.at[pl.ds(kk * _SP + off, 128)])

    return k(emb_t, tailp)


def _sc_fm(xp, et1, fc1, W, b):
    mesh = plsc.VectorSubcoreMesh(core_axis_name="c", subcore_axis_name="s")

    @functools.partial(
        pl.kernel,
        mesh=mesh,
        compiler_params=pltpu.CompilerParams(use_tc_tiling_on_sc=False),
        out_type=jax.ShapeDtypeStruct((_B,), jnp.float32),
        scratch_types=[
            pltpu.VMEM((_CH,), jnp.int32),
            pltpu.VMEM((_CH,), jnp.int32),
            pltpu.VMEM((_K, _CH), jnp.float32),
            pltpu.VMEM((_K, _CH), jnp.float32),
            pltpu.VMEM((_CH,), jnp.float32),
            pltpu.VMEM((_CH,), jnp.float32),
            pltpu.VMEM((_RCH,), jnp.float32),
            pltpu.VMEM((16,), jnp.float32),
            pltpu.VMEM((16,), jnp.float32),
            pltpu.SemaphoreType.DMA,
            pltpu.SemaphoreType.DMA,
        ],
    )
    def k(x_hbm, et_hbm, fc_hbm, w_hbm, b_hbm, o_hbm,
          idxb0, idxb1, ebuf0, ebuf1, fbuf0, fbuf1, obuf, wvm, bvm,
          sem0, sem1):
        pltpu.sync_copy(w_hbm, wvm)
        pltpu.sync_copy(b_hbm, bvm)
        w0 = wvm[...]
        b0 = bvm[...]
        wid = lax.axis_index("s") * _NC + lax.axis_index("c")
        base = wid * _PER_W
        rbase = wid * _RW
        sets = [(idxb0, ebuf0, fbuf0, sem0), (idxb1, ebuf1, fbuf1, sem1)]

        def fire(step, st):
            idxb, ebuf, fbuf, sem = st
            j0 = base + step * _CH
            pltpu.sync_copy(x_hbm.at[pl.ds(j0, _CH)], idxb)
            cps = []
            for kk in range(_K):
                src = et_hbm.at[pl.ds(kk * _SP, _N)]
                cps.append(pltpu.async_copy(src.at[idxb], ebuf.at[kk], sem))
            cps.append(pltpu.async_copy(fc_hbm.at[idxb], fbuf, sem))
            return cps

        cps = fire(0, sets[0])
        for step in range(_NSTEP):
            _, ebuf, fbuf, _ = sets[step % 2]
            cur_cps = cps
            if step + 1 < _NSTEP:
                cps = fire(step + 1, sets[(step + 1) % 2])
            for cp in cur_cps:
                cp.wait()

            @pl.loop(0, _RCH, step=16)
            def _(m):
                def kbody(kk, tacc):
                    s = ebuf[kk, pl.ds(m, 16)]
                    ss = s * s
                    for f in range(1, _F):
                        v = ebuf[kk, pl.ds(f * _RCH + m, 16)]
                        s = s + v
                        ss = ss + v * v
                    return tacc + s * s - ss

                t = lax.fori_loop(0, _K, kbody, jnp.zeros(16, jnp.float32))
                fcs = fbuf[pl.ds(m, 16)]
                for f in range(1, _F):
                    fcs = fcs + fbuf[pl.ds(f * _RCH + m, 16)]
                z = fcs * w0 + b0 + 0.5 * t
                obuf[pl.ds(m, 16)] = 1.0 / (1.0 + jnp.exp(-z))

            pltpu.sync_copy(obuf, o_hbm.at[pl.ds(rbase + step * _RCH, _RCH)])

    return k(xp, et1, fc1, W, b)


def kernel(x, emb_table, fc_table, W, b):
    tail = emb_table[_NBIG * _CC:, :]
    tailp = jnp.pad(tail, ((0, 128 - _REM), (0, 0))).T.reshape(_K * 128)
    et1 = _sc_relayout(emb_table.T, tailp)
    fc1 = fc_table.reshape(_N)
    xp = (x.reshape(_NW, _NSTEP, _RCH, _F)
          .transpose(0, 1, 3, 2)
          .reshape(_NIDX))
    w16 = jnp.broadcast_to(W.reshape(1), (16,))
    b16 = jnp.broadcast_to(b, (16,))
    return _sc_fm(xp, et1, fc1, w16, b16)
